# baseline (speedup 1.0000x reference)
import jax
import jax.numpy as jnp
from jax.experimental import pallas as pl

_ATTR = 8
_E_DIMS = 256
_BATCH = 4
_SEQ = 8192
_EROWS = 2048
_OROWS = _EROWS * _ATTR
_GRID = _BATCH * _SEQ // _OROWS


def _tc_broadcast(e):
    def body(e_ref, o_ref):
        x = e_ref[...]
        y = jnp.broadcast_to(
            x[:, None, :], (x.shape[0], _ATTR, _E_DIMS)
        ).reshape(x.shape[0] * _ATTR, _E_DIMS)
        for r in range(_OROWS // (x.shape[0] * _ATTR)):
            o_ref[pl.ds(r * x.shape[0] * _ATTR, x.shape[0] * _ATTR), :] = y

    erows = min(_EROWS, _SEQ // _ATTR)
    return pl.pallas_call(
        body,
        grid=(_GRID,),
        in_specs=[
            pl.BlockSpec((erows, _E_DIMS), lambda j: (j % max(1, _SEQ // (erows * _ATTR)), 0))
        ],
        out_specs=pl.BlockSpec((_OROWS, _E_DIMS), lambda j: (j, 0)),
        out_shape=jax.ShapeDtypeStruct((_BATCH * _SEQ, _E_DIMS), jnp.float32),
    )(e)


def kernel(x, E_absolute_position):
    del x
    return _tc_broadcast(E_absolute_position).reshape(_BATCH, _SEQ, _E_DIMS)
